# baseline (speedup 1.0000x reference)
import functools

import jax
import jax.numpy as jnp
from jax import lax
from jax.experimental import pallas as pl
from jax.experimental.pallas import tpu as pltpu
from jax.experimental.pallas import tpu_sc as plsc

NUM_CORES = 2
NUM_SUBCORES = 16
NUM_WORKERS = NUM_CORES * NUM_SUBCORES

DIM = 64

CHUNK = 128
CHUNKS_PER_GROUP = 4
GROUP = CHUNK * CHUNKS_PER_GROUP


def _embed_kernel(n_groups, table_hbm, idx_hbm, out_hbm, idx_v, rows_v, gsem):
  wid = lax.axis_index("s") * NUM_CORES + lax.axis_index("c")

  @pl.loop(0, n_groups)
  def _group(g):
    pltpu.sync_copy(idx_hbm.at[wid, g], idx_v)
    for j in range(CHUNKS_PER_GROUP):
      pltpu.async_copy(
          table_hbm.at[idx_v.at[j]],
          rows_v.at[pl.ds(j * CHUNK, CHUNK)],
          gsem,
      )
    for j in range(CHUNKS_PER_GROUP):
      pltpu.make_async_copy(
          table_hbm.at[idx_v.at[j]],
          rows_v.at[pl.ds(j * CHUNK, CHUNK)],
          gsem,
      ).wait()
    pltpu.sync_copy(rows_v, out_hbm.at[wid, g])


def kernel(x, table):
  batch, seq = x.shape
  total = batch * seq
  assert total % (NUM_WORKERS * GROUP) == 0
  n_groups = total // (NUM_WORKERS * GROUP)

  idx = x.astype(jnp.int32).reshape(NUM_WORKERS, n_groups, CHUNKS_PER_GROUP, CHUNK)

  mesh = plsc.VectorSubcoreMesh(core_axis_name="c", subcore_axis_name="s")
  run = pl.kernel(
      functools.partial(_embed_kernel, n_groups),
      out_type=jax.ShapeDtypeStruct((NUM_WORKERS, n_groups, GROUP, DIM), jnp.float32),
      mesh=mesh,
      scratch_types=[
          pltpu.VMEM((CHUNKS_PER_GROUP, CHUNK), jnp.int32),
          pltpu.VMEM((GROUP, DIM), jnp.float32),
          pltpu.SemaphoreType.DMA,
      ],
      compiler_params=pltpu.CompilerParams(use_tc_tiling_on_sc=False),
  )
  out = run(table, idx)
  return out.reshape(batch, seq, DIM)


# speedup vs baseline: 1.0435x; 1.0435x over previous
import functools

import jax
import jax.numpy as jnp
from jax import lax
from jax.experimental import pallas as pl
from jax.experimental.pallas import tpu as pltpu
from jax.experimental.pallas import tpu_sc as plsc

NUM_CORES = 2
NUM_SUBCORES = 16
NUM_WORKERS = NUM_CORES * NUM_SUBCORES

DIM = 64

CHUNK = 128
CHUNKS_PER_GROUP = 2
GROUP = CHUNK * CHUNKS_PER_GROUP
NBUF = 4


def _embed_kernel(n_groups, table_hbm, idx_hbm, out_hbm, idx_v, rows_v,
                  gsems, osems):
  wid = lax.axis_index("s") * NUM_CORES + lax.axis_index("c")

  pltpu.sync_copy(idx_hbm.at[wid], idx_v)

  def fire(b, g):
    for j in range(CHUNKS_PER_GROUP):
      pltpu.async_copy(
          table_hbm.at[idx_v.at[g, j]],
          rows_v.at[b, pl.ds(j * CHUNK, CHUNK)],
          gsems[b],
      )

  def drain(b, g):
    for j in range(CHUNKS_PER_GROUP):
      pltpu.make_async_copy(
          table_hbm.at[idx_v.at[g, j]],
          rows_v.at[b, pl.ds(j * CHUNK, CHUNK)],
          gsems[b],
      ).wait()

  def write(b, g):
    pltpu.async_copy(rows_v.at[b], out_hbm.at[wid, g], osems[b])

  def wait_write(b, g):
    pltpu.make_async_copy(rows_v.at[b], out_hbm.at[wid, g], osems[b]).wait()

  for b in range(NBUF):
    fire(b, b)

  @pl.loop(0, n_groups - NBUF, step=NBUF)
  def _steady(g0):
    for b in range(NBUF):
      g = g0 + b
      drain(b, g)
      write(b, g)
      wait_write(b, g)
      fire(b, g + NBUF)

  for b in range(NBUF):
    g = n_groups - NBUF + b
    drain(b, g)
    write(b, g)
  for b in range(NBUF):
    g = n_groups - NBUF + b
    wait_write(b, g)


def kernel(x, table):
  batch, seq = x.shape
  total = batch * seq
  assert total % (NUM_WORKERS * GROUP * NBUF) == 0
  n_groups = total // (NUM_WORKERS * GROUP)

  idx = x.astype(jnp.int32).reshape(
      NUM_WORKERS, n_groups, CHUNKS_PER_GROUP, CHUNK)

  mesh = plsc.VectorSubcoreMesh(core_axis_name="c", subcore_axis_name="s")
  run = pl.kernel(
      functools.partial(_embed_kernel, n_groups),
      out_type=jax.ShapeDtypeStruct(
          (NUM_WORKERS, n_groups, GROUP, DIM), jnp.float32),
      mesh=mesh,
      scratch_types=[
          pltpu.VMEM((n_groups, CHUNKS_PER_GROUP, CHUNK), jnp.int32),
          pltpu.VMEM((NBUF, GROUP, DIM), jnp.float32),
          [pltpu.SemaphoreType.DMA] * NBUF,
          [pltpu.SemaphoreType.DMA] * NBUF,
      ],
      compiler_params=pltpu.CompilerParams(use_tc_tiling_on_sc=False),
  )
  out = run(table, idx)
  return out.reshape(batch, seq, DIM)
